# baseline (speedup 1.0000x reference)
import jax
import jax.numpy as jnp
from jax import lax
from jax.experimental import pallas as pl
from jax.experimental.pallas import tpu as pltpu
from jax.experimental.pallas import tpu_sc as plsc

_NUM_FIELDS = 26
_FIELD_SIZE = 100000
_BATCH = 16384
_D = 16
_N = _BATCH * _NUM_FIELDS
_L = 16

_NC, _NS = 2, 16
_NW = _NC * _NS

_V = 2600000
_SB = 2048
_NFULL = _V // _SB
_TAILC = _V - _NFULL * _SB
_PHA = 39
_PHB0 = _NW * _PHA
_NPHB = _NFULL + 1 - _PHB0


def _tbody(tbl_t, out_lin, va, vb, rowbuf, sem_ia, sem_ib, sem_o):
    wid = lax.axis_index("s") * _NC + lax.axis_index("c")
    iota = lax.iota(jnp.int32, _L)
    iota16 = iota * _D

    def fire_in(sb, vbuf, sem, cols=_SB):
        c0 = sb * _SB
        pltpu.async_copy(
            tbl_t.at[pl.ds(0, 8), pl.ds(c0, cols)],
            vbuf.at[pl.ds(0, 8), pl.ds(0, cols)],
            sem,
        )
        pltpu.async_copy(
            tbl_t.at[pl.ds(8, 8), pl.ds(c0, cols)],
            vbuf.at[pl.ds(8, 8), pl.ds(0, cols)],
            sem,
        )

    def wait_in(vbuf, sem, cols=_SB):
        for h in range(2):
            pltpu.make_async_copy(
                tbl_t.at[pl.ds(h * 8, 8), pl.ds(0, cols)],
                vbuf.at[pl.ds(h * 8, 8), pl.ds(0, cols)],
                sem,
            ).wait()

    def transpose(vbuf, ng):
        def tg(g, _):
            base = iota16 + g * (_L * _D)
            for r in range(_D):
                v16 = vbuf[r, pl.ds(g * _L, _L)]
                plsc.store_scatter(rowbuf, [base + r], v16)
            return 0

        lax.fori_loop(0, ng, tg, 0)

    def fire_out(sb, n):
        pltpu.async_copy(
            rowbuf.at[pl.ds(0, n)], out_lin.at[pl.ds(sb * (_SB * _D), n)], sem_o
        )

    def wait_out(n):
        pltpu.make_async_copy(
            rowbuf.at[pl.ds(0, n)], out_lin.at[pl.ds(0, n)], sem_o
        ).wait()

    base = wid * _PHA
    fire_in(base, va, sem_ia)

    def pair(k, _):
        sb0 = base + 2 * k
        fire_in(sb0 + 1, vb, sem_ib)
        wait_in(va, sem_ia)

        @pl.when(k > 0)
        def _():
            wait_out(_SB * _D)

        transpose(va, _SB // _L)
        fire_out(sb0, _SB * _D)
        fire_in(sb0 + 2, va, sem_ia)
        wait_in(vb, sem_ib)
        wait_out(_SB * _D)
        transpose(vb, _SB // _L)
        fire_out(sb0 + 1, _SB * _D)
        return 0

    lax.fori_loop(0, (_PHA - 1) // 2, pair, 0)
    wait_in(va, sem_ia)
    wait_out(_SB * _D)
    transpose(va, _SB // _L)
    fire_out(base + _PHA - 1, _SB * _D)
    wait_out(_SB * _D)

    @pl.when(wid < _NPHB)
    def _phase_b():
        sb = _PHB0 + wid

        @pl.when(sb < _NFULL)
        def _in_full():
            fire_in(sb, va, sem_ia)
            wait_in(va, sem_ia)

        @pl.when(sb == _NFULL)
        def _in_tail():
            fire_in(sb, va, sem_ia, 9 * 128)
            wait_in(va, sem_ia, 9 * 128)

        @pl.when(sb < _NFULL)
        def _full():
            transpose(va, _SB // _L)
            fire_out(sb, _SB * _D)
            wait_out(_SB * _D)

        @pl.when(sb == _NFULL)
        def _tail():
            transpose(va, _TAILC // _L)
            fire_out(sb, _TAILC * _D)
            wait_out(_TAILC * _D)


def _relayout_table(table):
    mesh = plsc.VectorSubcoreMesh(core_axis_name="c", subcore_axis_name="s")
    return pl.kernel(
        _tbody,
        out_type=jax.ShapeDtypeStruct((_V * _D,), jnp.float32),
        mesh=mesh,
        compiler_params=pltpu.CompilerParams(
            use_tc_tiling_on_sc=True,
            needs_layout_passes=False,
            disable_bounds_checks=True,
        ),
        scratch_types=[
            pltpu.VMEM((_D, _SB), jnp.float32),
            pltpu.VMEM((_D, _SB), jnp.float32),
            pltpu.VMEM((_SB * _D,), jnp.float32),
            pltpu.SemaphoreType.DMA,
            pltpu.SemaphoreType.DMA,
            pltpu.SemaphoreType.DMA,
        ],
    )(table.T)


_BT = 128
_NBT = _BATCH // _BT
_BT_PER_W = _NBT // _NW
_CH = _BT * _NUM_FIELDS
_GROW = 128
_KG = _CH // _GROW
_NVEC = _CH // _L
_BLK = _NUM_FIELDS * _D * _BT
_OUT_FLAT = _NUM_FIELDS * 2 * _NBT * 8 * _BT


def _body(x_hbm, table_hbm, out_hbm, offs_v, idxa, idxb, rowsa, rowsb, blk_v,
          sema, semb, semo):
    wid = lax.axis_index("s") * _NC + lax.axis_index("c")

    iota = lax.iota(jnp.int32, _L)
    iota26 = iota * _NUM_FIELDS
    dvecs = [lax.broadcast(jnp.int32(d), (_L,)) for d in range(_D)]
    _H = _CH // 2

    def fill_vec(v, _):
        offs_v[pl.ds(v * _L, _L)] = ((v * _L + iota) % _NUM_FIELDS) * _FIELD_SIZE
        return 0

    lax.fori_loop(0, _NVEC, fill_vec, 0)

    def stage(bi, buf):
        sub = (wid * _BT_PER_W + bi) * _CH
        pltpu.sync_copy(x_hbm.at[pl.ds(sub, _CH)], buf)

        def add_vec(v, _):
            sl = pl.ds(v * _L, _L)
            buf[sl] = buf[sl] + offs_v[sl]
            return 0

        lax.fori_loop(0, _NVEC, add_vec, 0)

    def fire_gathers(buf):
        da, db = [], []
        for r in range(_H // _GROW):
            da.append(
                pltpu.async_copy(
                    table_hbm.at[buf.at[pl.ds(r * _GROW, _GROW)]],
                    rowsa.at[pl.ds(r * _GROW, _GROW)],
                    sema,
                )
            )
        for r in range(_H // _GROW):
            db.append(
                pltpu.async_copy(
                    table_hbm.at[buf.at[pl.ds(_H + r * _GROW, _GROW)]],
                    rowsb.at[pl.ds(r * _GROW, _GROW)],
                    semb,
                )
            )
        return da, db

    def reorder_half(rows, h):
        def reorder_f(f, _):
            fvec = iota26 + f
            rvecs = [fvec + g * (_L * _NUM_FIELDS) for g in range(_H // _GROW // 13 * 4)]
            fbase = f * (_D * _BT) + h * (_BT // 2)
            for d in range(_D):
                for g in range(4):
                    val = plsc.load_gather(rows, [rvecs[g], dvecs[d]])
                    blk_v[pl.ds(fbase + d * _BT + g * _L, _L)] = val
            return 0

        lax.fori_loop(0, _NUM_FIELDS, reorder_f, 0)

    def fire_outs(bt):
        descs = []
        for f in range(_NUM_FIELDS):
            for dh in range(2):
                src = (f * 2 + dh) * 1024
                descs.append(
                    pltpu.async_copy(
                        blk_v.at[pl.ds(src, 1024)],
                        out_hbm.at[
                            pl.ds(((f * 2 + dh) * _NBT + bt) * 1024, 1024)
                        ],
                        semo,
                    )
                )
        return descs

    ib = [idxa, idxb]
    stage(0, ib[0])
    gd = fire_gathers(ib[0])
    outd = None
    for bi in range(_BT_PER_W):
        if bi + 1 < _BT_PER_W:
            stage(bi + 1, ib[(bi + 1) % 2])
        for dsc in gd[0]:
            dsc.wait()
        if outd is not None:
            for dsc in outd:
                dsc.wait()
        reorder_half(rowsa, 0)
        for dsc in gd[1]:
            dsc.wait()
        reorder_half(rowsb, 1)
        outd = fire_outs(wid * _BT_PER_W + bi)
        if bi + 1 < _BT_PER_W:
            gd = fire_gathers(ib[(bi + 1) % 2])
    for dsc in outd:
        dsc.wait()


@jax.jit
def kernel(x, table):
    x_flat = x.reshape(_N)
    table_lin = _relayout_table(table).reshape(_V, _D)
    mesh = plsc.VectorSubcoreMesh(core_axis_name="c", subcore_axis_name="s")
    out_flat = pl.kernel(
        _body,
        out_type=jax.ShapeDtypeStruct((_OUT_FLAT,), jnp.float32),
        mesh=mesh,
        compiler_params=pltpu.CompilerParams(
            use_tc_tiling_on_sc=False, needs_layout_passes=False
        ),
        scratch_types=[
            pltpu.VMEM((_CH,), jnp.int32),
            pltpu.VMEM((_CH,), jnp.int32),
            pltpu.VMEM((_CH,), jnp.int32),
            pltpu.VMEM((_CH // 2, _D), jnp.float32),
            pltpu.VMEM((_CH // 2, _D), jnp.float32),
            pltpu.VMEM((_BLK,), jnp.float32),
            pltpu.SemaphoreType.DMA,
            pltpu.SemaphoreType.DMA,
            pltpu.SemaphoreType.DMA,
        ],
    )(x_flat, table_lin)
    out5 = out_flat.reshape(_NUM_FIELDS, 2, _NBT, 8, _BT)
    return out5.transpose(2, 4, 0, 1, 3).reshape(_BATCH, _NUM_FIELDS, _D)


# speedup vs baseline: 1.1455x; 1.1455x over previous
import jax
import jax.numpy as jnp
from jax import lax
from jax.experimental import pallas as pl
from jax.experimental.pallas import tpu as pltpu
from jax.experimental.pallas import tpu_sc as plsc

_NUM_FIELDS = 26
_FIELD_SIZE = 100000
_BATCH = 16384
_D = 16
_N = _BATCH * _NUM_FIELDS
_L = 16

_NC, _NS = 2, 16
_NW = _NC * _NS

_V = 2600000
_SB = 2048
_NFULL = _V // _SB
_TAILC = _V - _NFULL * _SB
_PHA = 39
_PHB0 = _NW * _PHA
_NPHB = _NFULL + 1 - _PHB0


def _tbody(tbl_t, out_lin, va, vb, rowbuf, sem_ia, sem_ib, sem_o):
    wid = lax.axis_index("s") * _NC + lax.axis_index("c")
    iota = lax.iota(jnp.int32, _L)
    iota16 = iota * _D

    def fire_in(sb, vbuf, sem, cols=_SB):
        c0 = sb * _SB
        pltpu.async_copy(
            tbl_t.at[pl.ds(0, 8), pl.ds(c0, cols)],
            vbuf.at[pl.ds(0, 8), pl.ds(0, cols)],
            sem,
        )
        pltpu.async_copy(
            tbl_t.at[pl.ds(8, 8), pl.ds(c0, cols)],
            vbuf.at[pl.ds(8, 8), pl.ds(0, cols)],
            sem,
        )

    def wait_in(vbuf, sem, cols=_SB):
        for h in range(2):
            pltpu.make_async_copy(
                tbl_t.at[pl.ds(h * 8, 8), pl.ds(0, cols)],
                vbuf.at[pl.ds(h * 8, 8), pl.ds(0, cols)],
                sem,
            ).wait()

    def transpose(vbuf, ng):
        def tg(g, _):
            base = iota16 + g * (_L * _D)
            for r in range(_D):
                v16 = vbuf[r, pl.ds(g * _L, _L)]
                plsc.store_scatter(rowbuf, [base + r], v16)
            return 0

        lax.fori_loop(0, ng, tg, 0)

    def fire_out(sb, n):
        pltpu.async_copy(
            rowbuf.at[pl.ds(0, n)], out_lin.at[pl.ds(sb * (_SB * _D), n)], sem_o
        )

    def wait_out(n):
        pltpu.make_async_copy(
            rowbuf.at[pl.ds(0, n)], out_lin.at[pl.ds(0, n)], sem_o
        ).wait()

    base = wid * _PHA
    fire_in(base, va, sem_ia)

    def pair(k, _):
        sb0 = base + 2 * k
        fire_in(sb0 + 1, vb, sem_ib)
        wait_in(va, sem_ia)

        @pl.when(k > 0)
        def _():
            wait_out(_SB * _D)

        transpose(va, _SB // _L)
        fire_out(sb0, _SB * _D)
        fire_in(sb0 + 2, va, sem_ia)
        wait_in(vb, sem_ib)
        wait_out(_SB * _D)
        transpose(vb, _SB // _L)
        fire_out(sb0 + 1, _SB * _D)
        return 0

    lax.fori_loop(0, (_PHA - 1) // 2, pair, 0)
    wait_in(va, sem_ia)
    wait_out(_SB * _D)
    transpose(va, _SB // _L)
    fire_out(base + _PHA - 1, _SB * _D)
    wait_out(_SB * _D)

    @pl.when(wid < _NPHB)
    def _phase_b():
        sb = _PHB0 + wid

        @pl.when(sb < _NFULL)
        def _in_full():
            fire_in(sb, va, sem_ia)
            wait_in(va, sem_ia)

        @pl.when(sb == _NFULL)
        def _in_tail():
            fire_in(sb, va, sem_ia, 9 * 128)
            wait_in(va, sem_ia, 9 * 128)

        @pl.when(sb < _NFULL)
        def _full():
            transpose(va, _SB // _L)
            fire_out(sb, _SB * _D)
            wait_out(_SB * _D)

        @pl.when(sb == _NFULL)
        def _tail():
            transpose(va, _TAILC // _L)
            fire_out(sb, _TAILC * _D)
            wait_out(_TAILC * _D)


def _relayout_table(table):
    mesh = plsc.VectorSubcoreMesh(core_axis_name="c", subcore_axis_name="s")
    return pl.kernel(
        _tbody,
        out_type=jax.ShapeDtypeStruct((_V * _D,), jnp.float32),
        mesh=mesh,
        compiler_params=pltpu.CompilerParams(
            use_tc_tiling_on_sc=True,
            needs_layout_passes=False,
            disable_bounds_checks=True,
        ),
        scratch_types=[
            pltpu.VMEM((_D, _SB), jnp.float32),
            pltpu.VMEM((_D, _SB), jnp.float32),
            pltpu.VMEM((_SB * _D,), jnp.float32),
            pltpu.SemaphoreType.DMA,
            pltpu.SemaphoreType.DMA,
            pltpu.SemaphoreType.DMA,
        ],
    )(table.T)


_BT = 128
_NBT = _BATCH // _BT
_BT_PER_W = _NBT // _NW
_CH = _BT * _NUM_FIELDS
_GROW = 128
_KG = _CH // _GROW
_NVEC = _CH // _L
_BLK = _NUM_FIELDS * _D * _BT
_OUT_FLAT = _NUM_FIELDS * 2 * _NBT * 8 * _BT


def _body(x_hbm, table_hbm, out_hbm, offs_v, idxa, idxb, rowsa, rowsb, blk_v,
          sema, semb, semo):
    wid = lax.axis_index("s") * _NC + lax.axis_index("c")

    iota = lax.iota(jnp.int32, _L)
    iota26 = iota * _NUM_FIELDS
    dvecs = [lax.broadcast(jnp.int32(d), (_L,)) for d in range(_D)]
    _H = _CH // 2

    def fill_vec(v, _):
        offs_v[pl.ds(v * _L, _L)] = ((v * _L + iota) % _NUM_FIELDS) * _FIELD_SIZE
        return 0

    lax.fori_loop(0, _NVEC, fill_vec, 0)

    def stage(bi, buf):
        sub = (wid * _BT_PER_W + bi) * _CH
        pltpu.sync_copy(x_hbm.at[pl.ds(sub, _CH)], buf)

        def add_vec(v, _):
            sl = pl.ds(v * _L, _L)
            buf[sl] = buf[sl] + offs_v[sl]
            return 0

        lax.fori_loop(0, _NVEC, add_vec, 0)

    def fire_gathers(buf):
        da, db = [], []
        for r in range(_H // _GROW):
            da.append(
                pltpu.async_copy(
                    table_hbm.at[buf.at[pl.ds(r * _GROW, _GROW)]],
                    rowsa.at[pl.ds(r * _GROW, _GROW)],
                    sema,
                )
            )
        for r in range(_H // _GROW):
            db.append(
                pltpu.async_copy(
                    table_hbm.at[buf.at[pl.ds(_H + r * _GROW, _GROW)]],
                    rowsb.at[pl.ds(r * _GROW, _GROW)],
                    semb,
                )
            )
        return da, db

    def reorder_half(rows, h):
        @plsc.parallel_loop(0, _NUM_FIELDS)
        def reorder_f(f):
            fvec = iota26 + f
            rvecs = [fvec + g * (_L * _NUM_FIELDS) for g in range(4)]
            fbase = f * (_D * _BT) + h * (_BT // 2)
            for d in range(_D):
                for g in range(4):
                    val = plsc.load_gather(rows, [rvecs[g], dvecs[d]])
                    blk_v[pl.ds(fbase + d * _BT + g * _L, _L)] = val

    def fire_outs(bt):
        descs = []
        for f in range(_NUM_FIELDS):
            for dh in range(2):
                src = (f * 2 + dh) * 1024
                descs.append(
                    pltpu.async_copy(
                        blk_v.at[pl.ds(src, 1024)],
                        out_hbm.at[
                            pl.ds(((f * 2 + dh) * _NBT + bt) * 1024, 1024)
                        ],
                        semo,
                    )
                )
        return descs

    ib = [idxa, idxb]
    stage(0, ib[0])
    gd = fire_gathers(ib[0])
    outd = None
    for bi in range(_BT_PER_W):
        if bi + 1 < _BT_PER_W:
            stage(bi + 1, ib[(bi + 1) % 2])
        for dsc in gd[0]:
            dsc.wait()
        if outd is not None:
            for dsc in outd:
                dsc.wait()
        reorder_half(rowsa, 0)
        for dsc in gd[1]:
            dsc.wait()
        reorder_half(rowsb, 1)
        outd = fire_outs(wid * _BT_PER_W + bi)
        if bi + 1 < _BT_PER_W:
            gd = fire_gathers(ib[(bi + 1) % 2])
    for dsc in outd:
        dsc.wait()


@jax.jit
def kernel(x, table):
    x_flat = x.reshape(_N)
    table_lin = _relayout_table(table).reshape(_V, _D)
    mesh = plsc.VectorSubcoreMesh(core_axis_name="c", subcore_axis_name="s")
    out_flat = pl.kernel(
        _body,
        out_type=jax.ShapeDtypeStruct((_OUT_FLAT,), jnp.float32),
        mesh=mesh,
        compiler_params=pltpu.CompilerParams(
            use_tc_tiling_on_sc=False, needs_layout_passes=False
        ),
        scratch_types=[
            pltpu.VMEM((_CH,), jnp.int32),
            pltpu.VMEM((_CH,), jnp.int32),
            pltpu.VMEM((_CH,), jnp.int32),
            pltpu.VMEM((_CH // 2, _D), jnp.float32),
            pltpu.VMEM((_CH // 2, _D), jnp.float32),
            pltpu.VMEM((_BLK,), jnp.float32),
            pltpu.SemaphoreType.DMA,
            pltpu.SemaphoreType.DMA,
            pltpu.SemaphoreType.DMA,
        ],
    )(x_flat, table_lin)
    out5 = out_flat.reshape(_NUM_FIELDS, 2, _NBT, 8, _BT)
    return out5.transpose(2, 4, 0, 1, 3).reshape(_BATCH, _NUM_FIELDS, _D)
